# Initial kernel scaffold; baseline (speedup 1.0000x reference)
import jax
import jax.numpy as jnp
from jax.experimental import pallas as pl


def kernel(x, enc1_w, enc1_b, mean_w, mean_b, logvar_w, logvar_b):
    raise NotImplementedError("write your pallas kernel here")


# speedup vs baseline: 1.2597x; 1.2597x over previous
import jax
import jax.numpy as jnp
from jax.experimental import pallas as pl
from jax.experimental.pallas import tpu as pltpu

_ROWS_PER_PROG = 32
_ACCS = 4


def _bag_kernel(x_vmem, table, out, x_smem, sem):
    S = x_vmem.shape[1]
    H = table.shape[2]
    cp = pltpu.make_async_copy(x_vmem, x_smem, sem)
    cp.start()
    cp.wait()

    def row_body(r, carry):
        accs = [jnp.zeros((H,), jnp.float32) for _ in range(_ACCS)]
        for t in range(S):
            idx = x_smem[r, t]
            accs[t % _ACCS] = accs[t % _ACCS] + table[idx, 0]
        total = accs[0] + accs[1]
        total = total + (accs[2] + accs[3])
        out[r, 0] = total
        return carry

    jax.lax.fori_loop(0, _ROWS_PER_PROG, row_body, 0)


def _head_kernel(hp, b1, mw, mb, lw, lb, mean_out, logvar_out):
    h = jnp.maximum(hp[...] + b1[...], 0.0)
    mean_out[...] = (
        jnp.dot(h, mw[...], preferred_element_type=jnp.float32) + mb[...]
    )
    logvar_out[...] = (
        jnp.dot(h, lw[...], preferred_element_type=jnp.float32) + lb[...]
    )


def kernel(x, enc1_w, enc1_b, mean_w, mean_b, logvar_w, logvar_b):
    B, S = x.shape
    H, V = enc1_w.shape
    O = mean_w.shape[0]
    R = _ROWS_PER_PROG
    num_progs = B // R

    x = x.astype(jnp.int32)
    table = enc1_w.T.reshape(V, 1, H)

    h_pre = pl.pallas_call(
        _bag_kernel,
        grid=(num_progs,),
        in_specs=[
            pl.BlockSpec((R, S), lambda i: (i, 0)),
            pl.BlockSpec((V, 1, H), lambda i: (0, 0, 0)),
        ],
        out_specs=pl.BlockSpec((R, 1, H), lambda i: (i, 0, 0)),
        out_shape=jax.ShapeDtypeStruct((B, 1, H), jnp.float32),
        scratch_shapes=[
            pltpu.SMEM((R, S), jnp.int32),
            pltpu.SemaphoreType.DMA,
        ],
        compiler_params=pltpu.CompilerParams(
            dimension_semantics=("parallel",),
            vmem_limit_bytes=48 * 1024 * 1024,
        ),
    )(x, table)

    h_pre = h_pre.reshape(B, H)

    rows_blk = B // 2
    mean, logvar = pl.pallas_call(
        _head_kernel,
        grid=(2,),
        in_specs=[
            pl.BlockSpec((rows_blk, H), lambda i: (i, 0)),
            pl.BlockSpec((1, H), lambda i: (0, 0)),
            pl.BlockSpec((H, O), lambda i: (0, 0)),
            pl.BlockSpec((1, O), lambda i: (0, 0)),
            pl.BlockSpec((H, O), lambda i: (0, 0)),
            pl.BlockSpec((1, O), lambda i: (0, 0)),
        ],
        out_specs=[
            pl.BlockSpec((rows_blk, O), lambda i: (i, 0)),
            pl.BlockSpec((rows_blk, O), lambda i: (i, 0)),
        ],
        out_shape=[
            jax.ShapeDtypeStruct((B, O), jnp.float32),
            jax.ShapeDtypeStruct((B, O), jnp.float32),
        ],
        compiler_params=pltpu.CompilerParams(
            dimension_semantics=("parallel",),
        ),
    )(
        h_pre,
        enc1_b.reshape(1, H),
        mean_w.T,
        mean_b.reshape(1, O),
        logvar_w.T,
        logvar_b.reshape(1, O),
    )
    return (mean, logvar)
